# Initial kernel scaffold; baseline (speedup 1.0000x reference)
import jax
import jax.numpy as jnp
from jax.experimental import pallas as pl


def kernel(latent_features, edge_data, history, is_subgoal, params):
    raise NotImplementedError("write your pallas kernel here")


# speedup vs baseline: 9.2512x; 9.2512x over previous
import jax
import jax.numpy as jnp
from jax import lax
from jax.experimental import pallas as pl
from jax.experimental.pallas import tpu as pltpu
from jax.experimental.pallas import tpu_sc as plsc

_N = 100000
_EH = 800000
_E = 2 * _EH
_NC, _NS = 2, 16
_NW = _NC * _NS
_CHUNK = 128
_SUP = 28
_OUT = 14
_PERW = _CHUNK * _SUP * _OUT
_EPAD = _NW * _PERW
_NT = 100096
_ZROWS = _NT // _NS
_CROWS = _N // _NS
_BR = 2000
_NB = _N // _BR
_EPS = 1e-5
_SLOPE = 0.1

_mesh = plsc.VectorSubcoreMesh(core_axis_name="c", subcore_axis_name="s")
_sc_params = pltpu.CompilerParams(use_tc_tiling_on_sc=False)


def _sc_count(dst4, zeros1):
  def body(dst_hbm, zeros_hbm, out_hbm, idx_v, ones_v, zcnt_v, cnt_sp, sem):
    cid = lax.axis_index("c")
    sid = lax.axis_index("s")
    wid = sid * _NC + cid
    for i in range(_CHUNK // 16):
      ones_v[pl.ds(i * 16, 16)] = jnp.ones((16,), jnp.float32)
    pltpu.sync_copy(zeros_hbm, zcnt_v)
    pltpu.sync_copy(zcnt_v, cnt_sp.at[pl.ds(sid * _ZROWS, _ZROWS)])
    plsc.subcore_barrier()

    def outer(o, _):
      pltpu.sync_copy(dst_hbm.at[wid, o], idx_v)

      def inner(j, _):
        pltpu.sync_copy(ones_v, cnt_sp.at[idx_v.at[j]], add=True)
        return 0

      lax.fori_loop(0, _SUP, inner, 0)
      return 0

    lax.fori_loop(0, _OUT, outer, 0)
    plsc.subcore_barrier()
    pltpu.sync_copy(cnt_sp.at[pl.ds(sid * _ZROWS, _ZROWS)], zcnt_v)
    pltpu.sync_copy(zcnt_v, out_hbm.at[cid, sid])

  f = pl.kernel(
      body,
      out_type=jax.ShapeDtypeStruct((_NC, _NS, _ZROWS), jnp.float32),
      mesh=_mesh,
      compiler_params=_sc_params,
      scratch_types=[
          pltpu.VMEM((_SUP, _CHUNK), jnp.int32),
          pltpu.VMEM((_CHUNK,), jnp.float32),
          pltpu.VMEM((_ZROWS,), jnp.float32),
          pltpu.VMEM_SHARED((_NT,), jnp.float32),
          pltpu.SemaphoreType.DMA,
      ],
  )
  return f(dst4, zeros1)


def _sc_agg(xt, src4, dst4, zeros8):
  def body(x_hbm, src_hbm, dst_hbm, zeros_hbm, out_hbm,
           isrc_v, idst_v, rows_v, zrow_v, agg_sp, sem):
    cid = lax.axis_index("c")
    sid = lax.axis_index("s")
    wid = sid * _NC + cid
    pltpu.sync_copy(zeros_hbm, zrow_v)
    pltpu.sync_copy(zrow_v, agg_sp.at[pl.ds(sid * _ZROWS, _ZROWS)])
    plsc.subcore_barrier()

    def outer(o, _):
      pltpu.sync_copy(src_hbm.at[wid, o], isrc_v)
      pltpu.sync_copy(dst_hbm.at[wid, o], idst_v)

      def inner(j, _):
        pltpu.async_copy(x_hbm.at[isrc_v.at[j]], rows_v, sem).wait()
        pltpu.sync_copy(rows_v, agg_sp.at[idst_v.at[j]], add=True)
        return 0

      lax.fori_loop(0, _SUP, inner, 0)
      return 0

    lax.fori_loop(0, _OUT, outer, 0)
    plsc.subcore_barrier()
    pltpu.sync_copy(agg_sp.at[pl.ds(sid * _CROWS, _CROWS)],
                    zrow_v.at[pl.ds(0, _CROWS)])
    pltpu.sync_copy(zrow_v.at[pl.ds(0, _CROWS)], out_hbm.at[cid, sid])

  f = pl.kernel(
      body,
      out_type=jax.ShapeDtypeStruct((_NC, _NS, _CROWS, 8), jnp.float32),
      mesh=_mesh,
      compiler_params=_sc_params,
      scratch_types=[
          pltpu.VMEM((_SUP, _CHUNK), jnp.int32),
          pltpu.VMEM((_SUP, _CHUNK), jnp.int32),
          pltpu.VMEM((_CHUNK, 8), jnp.float32),
          pltpu.VMEM((_ZROWS, 8), jnp.float32),
          pltpu.VMEM_SHARED((_NT, 8), jnp.float32),
          pltpu.SemaphoreType.DMA,
      ],
  )
  return f(xt, src4, dst4, zeros8)


def _acc_stats(i, y, st_ref):
  part = jnp.stack([jnp.sum(y, 0), jnp.sum(y * y, 0)])

  @pl.when(i == 0)
  def _():
    st_ref[...] = part

  @pl.when(i > 0)
  def _():
    st_ref[...] += part


def _bn_act(y, st, g, b):
  m = st[0] * (1.0 / _N)
  v = st[1] * (1.0 / _N) - m * m
  xh = (y - m[None, :]) * lax.rsqrt(v + _EPS)[None, :] * g + b
  return jnp.where(xh >= 0, xh, _SLOPE * xh)


def _whole(shape):
  return pl.BlockSpec(shape, lambda i: tuple(0 for _ in shape))


def _tc_mlp1(lf, h, s, W64, wh, ws, b1):
  def body(lf_ref, h_ref, s_ref, w_ref, wh_ref, ws_ref, b_ref, y_ref, st_ref):
    i = pl.program_id(0)
    y = jnp.dot(lf_ref[...], w_ref[...].T, preferred_element_type=jnp.float32)
    y = y + h_ref[...] * wh_ref[...] + s_ref[...] * ws_ref[...] + b_ref[...]
    y_ref[...] = y
    _acc_stats(i, y, st_ref)

  return pl.pallas_call(
      body,
      grid=(_NB,),
      in_specs=[
          pl.BlockSpec((_BR, 64), lambda i: (i, 0)),
          pl.BlockSpec((_BR, 1), lambda i: (i, 0)),
          pl.BlockSpec((_BR, 1), lambda i: (i, 0)),
          _whole((32, 64)), _whole((1, 32)), _whole((1, 32)), _whole((1, 32)),
      ],
      out_specs=[
          pl.BlockSpec((_BR, 32), lambda i: (i, 0)),
          _whole((2, 32)),
      ],
      out_shape=[
          jax.ShapeDtypeStruct((_N, 32), jnp.float32),
          jax.ShapeDtypeStruct((2, 32), jnp.float32),
      ],
  )(lf, h, s, W64, wh, ws, b1)


def _tc_mid(y_in, st_in, g, bbn, W, b, din, dout):
  def body(y_ref, st_ref_in, g_ref, bbn_ref, w_ref, b_ref, y_ref_out, st_ref):
    i = pl.program_id(0)
    x = _bn_act(y_ref[...], st_ref_in[...], g_ref[...], bbn_ref[...])
    y = jnp.dot(x, w_ref[...].T, preferred_element_type=jnp.float32) + b_ref[...]
    y_ref_out[...] = y
    _acc_stats(i, y, st_ref)

  return pl.pallas_call(
      body,
      grid=(_NB,),
      in_specs=[
          pl.BlockSpec((_BR, din), lambda i: (i, 0)),
          _whole((2, din)), _whole((1, din)), _whole((1, din)),
          _whole((dout, din)), _whole((1, dout)),
      ],
      out_specs=[
          pl.BlockSpec((_BR, dout), lambda i: (i, 0)),
          _whole((2, dout)),
      ],
      out_shape=[
          jax.ShapeDtypeStruct((_N, dout), jnp.float32),
          jax.ShapeDtypeStruct((2, dout), jnp.float32),
      ],
  )(y_in, st_in, g, bbn, W, b)


def _tc_bnact(y_in, st_in, g, bbn, d):
  def body(y_ref, st_ref, g_ref, b_ref, x_ref):
    x_ref[...] = _bn_act(y_ref[...], st_ref[...], g_ref[...], b_ref[...])

  return pl.pallas_call(
      body,
      grid=(_NB,),
      in_specs=[
          pl.BlockSpec((_BR, d), lambda i: (i, 0)),
          _whole((2, d)), _whole((1, d)), _whole((1, d)),
      ],
      out_specs=pl.BlockSpec((_BR, d), lambda i: (i, 0)),
      out_shape=jax.ShapeDtypeStruct((_N, d), jnp.float32),
  )(y_in, st_in, g, bbn)


def _tc_combine(agg2, cnt2, x, Wl, bl, Wr):
  def body(a_ref, c_ref, x_ref, wl_ref, bl_ref, wr_ref, z_ref, st_ref):
    i = pl.program_id(0)
    a = a_ref[0] + a_ref[1]
    c = c_ref[0] + c_ref[1]
    mean = a / jnp.maximum(c, 1.0)
    z = (jnp.dot(mean, wl_ref[...].T, preferred_element_type=jnp.float32)
         + bl_ref[...]
         + jnp.dot(x_ref[...], wr_ref[...].T,
                   preferred_element_type=jnp.float32))
    z_ref[...] = z
    _acc_stats(i, z, st_ref)

  return pl.pallas_call(
      body,
      grid=(_NB,),
      in_specs=[
          pl.BlockSpec((2, _BR, 8), lambda i: (0, i, 0)),
          pl.BlockSpec((2, _BR, 1), lambda i: (0, i, 0)),
          pl.BlockSpec((_BR, 8), lambda i: (i, 0)),
          _whole((8, 8)), _whole((1, 8)), _whole((8, 8)),
      ],
      out_specs=[
          pl.BlockSpec((_BR, 8), lambda i: (i, 0)),
          _whole((2, 8)),
      ],
      out_shape=[
          jax.ShapeDtypeStruct((_N, 8), jnp.float32),
          jax.ShapeDtypeStruct((2, 8), jnp.float32),
      ],
  )(agg2, cnt2, x, Wl, bl, Wr)


def _tc_cls(x, Wc, bc):
  def body(x_ref, w_ref, b_ref, o_ref):
    o_ref[...] = jnp.dot(x_ref[...], w_ref[...].T,
                         preferred_element_type=jnp.float32) + b_ref[...]

  return pl.pallas_call(
      body,
      grid=(_NB,),
      in_specs=[
          pl.BlockSpec((_BR, 8), lambda i: (i, 0)),
          _whole((3, 8)), _whole((1, 3)),
      ],
      out_specs=pl.BlockSpec((_BR, 3), lambda i: (i, 0)),
      out_shape=jax.ShapeDtypeStruct((_N, 3), jnp.float32),
  )(x, Wc, bc)


def kernel(latent_features, edge_data, history, is_subgoal, params):
  p = params
  lf = latent_features.astype(jnp.float32)
  h = history.astype(jnp.float32).reshape(_N, 1)
  s = is_subgoal.astype(jnp.float32).reshape(_N, 1)

  e0 = edge_data[0].astype(jnp.int32)
  e1 = edge_data[1].astype(jnp.int32)
  src = jnp.concatenate([e0, e1])
  dst = jnp.concatenate([e1, e0])
  pad = jnp.full((_EPAD - _E,), _N, jnp.int32)
  src4 = jnp.concatenate([src, pad]).reshape(_NW, _OUT, _SUP, _CHUNK)
  dst4 = jnp.concatenate([dst, pad]).reshape(_NW, _OUT, _SUP, _CHUNK)
  zeros1 = jnp.zeros((_ZROWS,), jnp.float32)
  zeros8 = jnp.zeros((_ZROWS, 8), jnp.float32)

  cnt_parts = _sc_count(dst4, zeros1)
  cnt2 = cnt_parts.reshape(_NC, _NT)[:, :_N]
  cnt2 = cnt2.reshape(_NC, _N, 1)

  W1 = p['fc1_W']
  y1, st1 = _tc_mlp1(lf, h, s, W1[:, :64],
                     W1[:, 64].reshape(1, 32), W1[:, 65].reshape(1, 32),
                     p['fc1_b'].reshape(1, 32))
  y2, st2 = _tc_mid(y1, st1, p['fc1bn_g'].reshape(1, 32),
                    p['fc1bn_b'].reshape(1, 32), p['fc2_W'],
                    p['fc2_b'].reshape(1, 16), 32, 16)
  y3, st3 = _tc_mid(y2, st2, p['fc2bn_g'].reshape(1, 16),
                    p['fc2bn_b'].reshape(1, 16), p['fc3_W'],
                    p['fc3_b'].reshape(1, 8), 16, 8)
  x = _tc_bnact(y3, st3, p['fc3bn_g'].reshape(1, 8),
                p['fc3bn_b'].reshape(1, 8), 8)

  for k in (1, 2, 3):
    xt = jnp.concatenate([x, jnp.zeros((_NT - _N, 8), jnp.float32)])
    agg_parts = _sc_agg(xt, src4, dst4, zeros8)
    agg2 = agg_parts.reshape(_NC, _N, 8)
    z, stz = _tc_combine(agg2, cnt2, x,
                         p['conv%d_Wl' % k], p['conv%d_bl' % k].reshape(1, 8),
                         p['conv%d_Wr' % k])
    x = _tc_bnact(z, stz, p['conv%dbn_g' % k].reshape(1, 8),
                  p['conv%dbn_b' % k].reshape(1, 8), 8)

  return _tc_cls(x, p['cls_W'], p['cls_b'].reshape(1, 3))


# speedup vs baseline: 13.2480x; 1.4320x over previous
import jax
import jax.numpy as jnp
from jax import lax
from jax.experimental import pallas as pl
from jax.experimental.pallas import tpu as pltpu
from jax.experimental.pallas import tpu_sc as plsc

_N = 100000
_EH = 800000
_E = 2 * _EH
_NC, _NS = 2, 16
_NW = _NC * _NS
_CHUNK = 128
_GRP = 8
_SUP = 56
_OUT = 7
_PERW = _CHUNK * _SUP * _OUT
_EPAD = _NW * _PERW
_NT = 100096
_ZROWS = _NT // _NS
_CROWS = _N // _NS
_BR = 2000
_NB = _N // _BR
_EPS = 1e-5
_SLOPE = 0.1

_mesh = plsc.VectorSubcoreMesh(core_axis_name="c", subcore_axis_name="s")
_sc_params = pltpu.CompilerParams(use_tc_tiling_on_sc=False)


def _sc_count(dst4, zeros1):
  def body(dst_hbm, zeros_hbm, out_hbm, idx_v, ones_v, zcnt_v, cnt_sp, sem):
    cid = lax.axis_index("c")
    sid = lax.axis_index("s")
    wid = sid * _NC + cid
    for i in range(_CHUNK // 16):
      ones_v[pl.ds(i * 16, 16)] = jnp.ones((16,), jnp.float32)
    pltpu.sync_copy(zeros_hbm, zcnt_v)
    pltpu.sync_copy(zcnt_v, cnt_sp.at[pl.ds(sid * _ZROWS, _ZROWS)])
    plsc.subcore_barrier()

    def outer(o, _):
      pltpu.sync_copy(dst_hbm.at[wid, o], idx_v)

      def inner(p, _):
        sds = [
            pltpu.async_copy(ones_v, cnt_sp.at[idx_v.at[p * _GRP + b]], sem,
                             add=True)
            for b in range(_GRP)
        ]
        for sd in sds:
          sd.wait()
        return 0

      lax.fori_loop(0, _SUP // _GRP, inner, 0)
      return 0

    lax.fori_loop(0, _OUT, outer, 0)
    plsc.subcore_barrier()
    pltpu.sync_copy(cnt_sp.at[pl.ds(sid * _ZROWS, _ZROWS)], zcnt_v)
    pltpu.sync_copy(zcnt_v, out_hbm.at[cid, sid])

  f = pl.kernel(
      body,
      out_type=jax.ShapeDtypeStruct((_NC, _NS, _ZROWS), jnp.float32),
      mesh=_mesh,
      compiler_params=_sc_params,
      scratch_types=[
          pltpu.VMEM((_SUP, _CHUNK), jnp.int32),
          pltpu.VMEM((_CHUNK,), jnp.float32),
          pltpu.VMEM((_ZROWS,), jnp.float32),
          pltpu.VMEM_SHARED((_NT,), jnp.float32),
          pltpu.SemaphoreType.DMA,
      ],
  )
  return f(dst4, zeros1)


def _sc_agg(xt, src4, dst4, zeros8):
  def body(x_hbm, src_hbm, dst_hbm, zeros_hbm, out_hbm,
           isrc_v, idst_v, rows_v, zrow_v, agg_sp, gsem, ssem):
    cid = lax.axis_index("c")
    sid = lax.axis_index("s")
    wid = sid * _NC + cid
    pltpu.sync_copy(zeros_hbm, zrow_v)
    pltpu.sync_copy(zrow_v, agg_sp.at[pl.ds(sid * _ZROWS, _ZROWS)])
    plsc.subcore_barrier()

    def outer(o, _):
      pltpu.sync_copy(src_hbm.at[wid, o], isrc_v)
      pltpu.sync_copy(dst_hbm.at[wid, o], idst_v)

      def inner(p, _):
        j0 = p * _GRP
        gds = [
            pltpu.async_copy(x_hbm.at[isrc_v.at[j0 + b]], rows_v.at[b], gsem)
            for b in range(_GRP)
        ]
        sds = []
        for b in range(_GRP):
          gds[b].wait()
          sds.append(
              pltpu.async_copy(rows_v.at[b], agg_sp.at[idst_v.at[j0 + b]],
                               ssem, add=True))
        for sd in sds:
          sd.wait()
        return 0

      lax.fori_loop(0, _SUP // _GRP, inner, 0)
      return 0

    lax.fori_loop(0, _OUT, outer, 0)
    plsc.subcore_barrier()
    pltpu.sync_copy(agg_sp.at[pl.ds(sid * _CROWS, _CROWS)],
                    zrow_v.at[pl.ds(0, _CROWS)])
    pltpu.sync_copy(zrow_v.at[pl.ds(0, _CROWS)], out_hbm.at[cid, sid])

  f = pl.kernel(
      body,
      out_type=jax.ShapeDtypeStruct((_NC, _NS, _CROWS, 8), jnp.float32),
      mesh=_mesh,
      compiler_params=_sc_params,
      scratch_types=[
          pltpu.VMEM((_SUP, _CHUNK), jnp.int32),
          pltpu.VMEM((_SUP, _CHUNK), jnp.int32),
          pltpu.VMEM((_GRP, _CHUNK, 8), jnp.float32),
          pltpu.VMEM((_ZROWS, 8), jnp.float32),
          pltpu.VMEM_SHARED((_NT, 8), jnp.float32),
          pltpu.SemaphoreType.DMA,
          pltpu.SemaphoreType.DMA,
      ],
  )
  return f(xt, src4, dst4, zeros8)


def _acc_stats(i, y, st_ref):
  part = jnp.stack([jnp.sum(y, 0), jnp.sum(y * y, 0)])

  @pl.when(i == 0)
  def _():
    st_ref[...] = part

  @pl.when(i > 0)
  def _():
    st_ref[...] += part


def _bn_act(y, st, g, b):
  m = st[0] * (1.0 / _N)
  v = st[1] * (1.0 / _N) - m * m
  xh = (y - m[None, :]) * lax.rsqrt(v + _EPS)[None, :] * g + b
  return jnp.where(xh >= 0, xh, _SLOPE * xh)


def _whole(shape):
  return pl.BlockSpec(shape, lambda i: tuple(0 for _ in shape))


def _row_spec(d):
  return pl.BlockSpec((_BR, d), lambda i: (i, 0))


def _tc_mlp1(lf, h, s, W64, wh, ws, b1):
  def body(lf_ref, h_ref, s_ref, w_ref, wh_ref, ws_ref, b_ref, y_ref, st_ref):
    i = pl.program_id(0)
    y = jnp.dot(lf_ref[...], w_ref[...].T, preferred_element_type=jnp.float32)
    y = y + h_ref[...] * wh_ref[...] + s_ref[...] * ws_ref[...] + b_ref[...]
    y_ref[...] = y
    _acc_stats(i, y, st_ref)

  return pl.pallas_call(
      body,
      grid=(_NB,),
      in_specs=[
          _row_spec(64), _row_spec(1), _row_spec(1),
          _whole((32, 64)), _whole((1, 32)), _whole((1, 32)), _whole((1, 32)),
      ],
      out_specs=[_row_spec(32), _whole((2, 32))],
      out_shape=[
          jax.ShapeDtypeStruct((_N, 32), jnp.float32),
          jax.ShapeDtypeStruct((2, 32), jnp.float32),
      ],
  )(lf, h, s, W64, wh, ws, b1)


def _tc_mid(y_in, st_in, g, bbn, W, b, din, dout):
  def body(y_ref, st_ref_in, g_ref, bbn_ref, w_ref, b_ref, y_ref_out, st_ref):
    i = pl.program_id(0)
    x = _bn_act(y_ref[...], st_ref_in[...], g_ref[...], bbn_ref[...])
    y = jnp.dot(x, w_ref[...].T, preferred_element_type=jnp.float32) + b_ref[...]
    y_ref_out[...] = y
    _acc_stats(i, y, st_ref)

  return pl.pallas_call(
      body,
      grid=(_NB,),
      in_specs=[
          _row_spec(din),
          _whole((2, din)), _whole((1, din)), _whole((1, din)),
          _whole((dout, din)), _whole((1, dout)),
      ],
      out_specs=[_row_spec(dout), _whole((2, dout))],
      out_shape=[
          jax.ShapeDtypeStruct((_N, dout), jnp.float32),
          jax.ShapeDtypeStruct((2, dout), jnp.float32),
      ],
  )(y_in, st_in, g, bbn, W, b)


def _tc_bnact_pad(y_in, st_in, g, bbn):
  def body(y_ref, st_ref, g_ref, b_ref, x_ref):
    x_ref[...] = _bn_act(y_ref[...], st_ref[...], g_ref[...], b_ref[...])

  return pl.pallas_call(
      body,
      grid=(_NB,),
      in_specs=[
          _row_spec(8),
          _whole((2, 8)), _whole((1, 8)), _whole((1, 8)),
      ],
      out_specs=_row_spec(8),
      out_shape=jax.ShapeDtypeStruct((_NT, 8), jnp.float32),
  )(y_in, st_in, g, bbn)


def _tc_combine(agg_parts, cnt3, x, Wl, bl, Wr):
  def body(a_ref, c_ref, x_ref, wl_ref, bl_ref, wr_ref, z_ref, st_ref):
    i = pl.program_id(0)
    a = a_ref[0] + a_ref[1]
    c = c_ref[0] + c_ref[1]
    mean = a / jnp.maximum(c, 1.0)
    z = (jnp.dot(mean, wl_ref[...].T, preferred_element_type=jnp.float32)
         + bl_ref[...]
         + jnp.dot(x_ref[...], wr_ref[...].T,
                   preferred_element_type=jnp.float32))
    z_ref[...] = z
    _acc_stats(i, z, st_ref)

  return pl.pallas_call(
      body,
      grid=(_NB,),
      in_specs=[
          pl.BlockSpec((2, _BR, 8), lambda i: (0, i, 0)),
          pl.BlockSpec((2, _BR, 1), lambda i: (0, i, 0)),
          _row_spec(8),
          _whole((8, 8)), _whole((1, 8)), _whole((8, 8)),
      ],
      out_specs=[_row_spec(8), _whole((2, 8))],
      out_shape=[
          jax.ShapeDtypeStruct((_N, 8), jnp.float32),
          jax.ShapeDtypeStruct((2, 8), jnp.float32),
      ],
  )(agg_parts, cnt3, x, Wl, bl, Wr)


def _tc_bnact_cls(y_in, st_in, g, bbn, Wc, bc):
  def body(y_ref, st_ref, g_ref, b_ref, w_ref, bc_ref, o_ref):
    x = _bn_act(y_ref[...], st_ref[...], g_ref[...], b_ref[...])
    o_ref[...] = jnp.dot(x, w_ref[...].T,
                         preferred_element_type=jnp.float32) + bc_ref[...]

  return pl.pallas_call(
      body,
      grid=(_NB,),
      in_specs=[
          _row_spec(8),
          _whole((2, 8)), _whole((1, 8)), _whole((1, 8)),
          _whole((3, 8)), _whole((1, 3)),
      ],
      out_specs=pl.BlockSpec((_BR, 3), lambda i: (i, 0)),
      out_shape=jax.ShapeDtypeStruct((_N, 3), jnp.float32),
  )(y_in, st_in, g, bbn, Wc, bc)


def kernel(latent_features, edge_data, history, is_subgoal, params):
  p = params
  lf = latent_features.astype(jnp.float32)
  h = history.astype(jnp.float32).reshape(_N, 1)
  s = is_subgoal.astype(jnp.float32).reshape(_N, 1)

  e0 = edge_data[0].astype(jnp.int32)
  e1 = edge_data[1].astype(jnp.int32)
  src = jnp.concatenate([e0, e1])
  dst = jnp.concatenate([e1, e0])
  pad = jnp.full((_EPAD - _E,), _N, jnp.int32)
  src4 = jnp.concatenate([src, pad]).reshape(_NW, _OUT, _SUP, _CHUNK)
  dst4 = jnp.concatenate([dst, pad]).reshape(_NW, _OUT, _SUP, _CHUNK)
  zeros1 = jnp.zeros((_ZROWS,), jnp.float32)
  zeros8 = jnp.zeros((_ZROWS, 8), jnp.float32)

  cnt_parts = _sc_count(dst4, zeros1)
  cnt3 = cnt_parts.reshape(_NC, _NT, 1)

  W1 = p['fc1_W']
  y1, st1 = _tc_mlp1(lf, h, s, W1[:, :64],
                     W1[:, 64].reshape(1, 32), W1[:, 65].reshape(1, 32),
                     p['fc1_b'].reshape(1, 32))
  y2, st2 = _tc_mid(y1, st1, p['fc1bn_g'].reshape(1, 32),
                    p['fc1bn_b'].reshape(1, 32), p['fc2_W'],
                    p['fc2_b'].reshape(1, 16), 32, 16)
  y3, st3 = _tc_mid(y2, st2, p['fc2bn_g'].reshape(1, 16),
                    p['fc2bn_b'].reshape(1, 16), p['fc3_W'],
                    p['fc3_b'].reshape(1, 8), 16, 8)
  xt = _tc_bnact_pad(y3, st3, p['fc3bn_g'].reshape(1, 8),
                     p['fc3bn_b'].reshape(1, 8))

  z, stz = None, None
  for k in (1, 2, 3):
    agg_parts = _sc_agg(xt, src4, dst4, zeros8)
    agg2 = agg_parts.reshape(_NC, _N, 8)
    z, stz = _tc_combine(agg2, cnt3, xt, p['conv%d_Wl' % k],
                         p['conv%d_bl' % k].reshape(1, 8), p['conv%d_Wr' % k])
    if k < 3:
      xt = _tc_bnact_pad(z, stz, p['conv%dbn_g' % k].reshape(1, 8),
                         p['conv%dbn_b' % k].reshape(1, 8))

  return _tc_bnact_cls(z, stz, p['conv3bn_g'].reshape(1, 8),
                       p['conv3bn_b'].reshape(1, 8),
                       p['cls_W'], p['cls_b'].reshape(1, 3))


# speedup vs baseline: 22.2963x; 1.6830x over previous
import jax
import jax.numpy as jnp
from jax import lax
from jax.experimental import pallas as pl
from jax.experimental.pallas import tpu as pltpu
from jax.experimental.pallas import tpu_sc as plsc

_N = 100000
_EH = 800000
_NC, _NS = 2, 16
_NW = _NC * _NS
_CHUNK = 128
_GRP = 4
_SUP = 28
_OUT = 7
_PERW = _CHUNK * _SUP * _OUT
_EHPAD = _NW * _PERW
_NT = 100096
_ZROWS = _NT // _NS
_SROWS = _NT // 16
_VROWS = _N // 16
_BR = 2000
_NB = _N // _BR
_EPS = 1e-5
_SLOPE = 0.1

_mesh = plsc.VectorSubcoreMesh(core_axis_name="c", subcore_axis_name="s")
_sc_params = pltpu.CompilerParams(use_tc_tiling_on_sc=False,
                                  needs_layout_passes=False)


def _sc_recip8(epad, zeros1):
  def body(e_hbm, zeros_hbm, out_hbm, ia_v, ib_v, ones_v, zcnt_v, rexp_v,
           cnt_sp, sem):
    cid = lax.axis_index("c")
    sid = lax.axis_index("s")

    @pl.when(cid == 0)
    def _():
      for i in range(_CHUNK // 16):
        ones_v[pl.ds(i * 16, 16)] = jnp.ones((16,), jnp.float32)
      pltpu.sync_copy(zeros_hbm, zcnt_v)
      pltpu.sync_copy(zcnt_v, cnt_sp.at[pl.ds(sid * _ZROWS, _ZROWS)])
      plsc.subcore_barrier()

      def outer(o, _):
        blk = sid * 2 + o // _OUT
        oo = o % _OUT
        pltpu.sync_copy(e_hbm.at[0, blk, oo], ia_v)
        pltpu.sync_copy(e_hbm.at[1, blk, oo], ib_v)

        def inner(p, _):
          j0 = p * _GRP
          sds = []
          for b in range(_GRP):
            sds.append(pltpu.async_copy(
                ones_v, cnt_sp.at[ia_v.at[j0 + b]], sem, add=True))
            sds.append(pltpu.async_copy(
                ones_v, cnt_sp.at[ib_v.at[j0 + b]], sem, add=True))
          for sd in sds:
            sd.wait()
          return 0

        lax.fori_loop(0, _SUP // _GRP, inner, 0)
        return 0

      lax.fori_loop(0, 2 * _OUT, outer, 0)
      plsc.subcore_barrier()
      pltpu.sync_copy(cnt_sp.at[pl.ds(sid * _ZROWS, _ZROWS)], zcnt_v)

      def expand(i, _):
        c = zcnt_v[pl.ds(i * 16, 16)]
        r = 1.0 / jnp.maximum(c, 1.0)
        rows = i * 16 + lax.iota(jnp.int32, 16)
        for f in range(8):
          plsc.store_scatter(rexp_v, [rows, jnp.full((16,), f, jnp.int32)], r)
        return 0

      lax.fori_loop(0, _ZROWS // 16, expand, 0)
      pltpu.sync_copy(rexp_v, out_hbm.at[sid])

  f = pl.kernel(
      body,
      out_type=jax.ShapeDtypeStruct((_NS, _ZROWS, 8), jnp.float32),
      mesh=_mesh,
      compiler_params=_sc_params,
      scratch_types=[
          pltpu.VMEM((_SUP, _CHUNK), jnp.int32),
          pltpu.VMEM((_SUP, _CHUNK), jnp.int32),
          pltpu.VMEM((_CHUNK,), jnp.float32),
          pltpu.VMEM((_ZROWS,), jnp.float32),
          pltpu.VMEM((_ZROWS, 8), jnp.float32),
          pltpu.VMEM_SHARED((_NT,), jnp.float32),
          pltpu.SemaphoreType.DMA,
      ],
  )
  return f(epad, zeros1)


def _sc_agg(xt, epad, zeros8):
  def body(x_hbm, e_hbm, zeros_hbm, out_hbm,
           ia_v, ib_v, rows_v, zrow_v, agg_sp, gsem, ssem):
    cid = lax.axis_index("c")
    sid = lax.axis_index("s")
    wid = sid * _NC + cid
    pltpu.sync_copy(zeros_hbm, zrow_v)
    pltpu.sync_copy(zrow_v, agg_sp.at[pl.ds(sid * _ZROWS, _ZROWS)])
    plsc.subcore_barrier()

    def outer(o, _):
      pltpu.sync_copy(e_hbm.at[0, wid, o], ia_v)
      pltpu.sync_copy(e_hbm.at[1, wid, o], ib_v)

      def inner(p, _):
        j0 = p * _GRP
        gds = []
        for b in range(_GRP):
          gds.append(pltpu.async_copy(
              x_hbm.at[ia_v.at[j0 + b]], rows_v.at[b], gsem))
          gds.append(pltpu.async_copy(
              x_hbm.at[ib_v.at[j0 + b]], rows_v.at[_GRP + b], gsem))
        sds = []
        for b in range(_GRP):
          gds[2 * b].wait()
          sds.append(pltpu.async_copy(
              rows_v.at[b], agg_sp.at[ib_v.at[j0 + b]], ssem, add=True))
          gds[2 * b + 1].wait()
          sds.append(pltpu.async_copy(
              rows_v.at[_GRP + b], agg_sp.at[ia_v.at[j0 + b]], ssem,
              add=True))
        for sd in sds:
          sd.wait()
        return 0

      lax.fori_loop(0, _SUP // _GRP, inner, 0)
      return 0

    lax.fori_loop(0, _OUT, outer, 0)
    plsc.subcore_barrier()
    pltpu.sync_copy(agg_sp.at[pl.ds(sid * _ZROWS, _ZROWS)], zrow_v)
    pltpu.sync_copy(zrow_v, out_hbm.at[cid, sid])

  f = pl.kernel(
      body,
      out_type=jax.ShapeDtypeStruct((_NC, _NS, _ZROWS, 8), jnp.float32),
      mesh=_mesh,
      compiler_params=_sc_params,
      scratch_types=[
          pltpu.VMEM((_SUP, _CHUNK), jnp.int32),
          pltpu.VMEM((_SUP, _CHUNK), jnp.int32),
          pltpu.VMEM((2 * _GRP, _CHUNK, 8), jnp.float32),
          pltpu.VMEM((_ZROWS, 8), jnp.float32),
          pltpu.VMEM_SHARED((_NT, 8), jnp.float32),
          pltpu.SemaphoreType.DMA,
          pltpu.SemaphoreType.DMA,
      ],
  )
  return f(xt, epad, zeros8)


def _acc_stats(i, y, st_ref):
  part = jnp.stack([jnp.sum(y, 0), jnp.sum(y * y, 0)])

  @pl.when(i == 0)
  def _():
    st_ref[...] = part

  @pl.when(i > 0)
  def _():
    st_ref[...] += part


def _bn_act(y, m, v, g, b):
  xh = (y - m) * lax.rsqrt(v + _EPS) * g + b
  return jnp.where(xh >= 0, xh, _SLOPE * xh)


def _mv(st):
  m = st[0] * (1.0 / _N)
  v = st[1] * (1.0 / _N) - m * m
  return m[None, :], v[None, :]


def _whole(shape):
  return pl.BlockSpec(shape, lambda i: tuple(0 for _ in shape))


def _row_spec(d):
  return pl.BlockSpec((_BR, d), lambda i: (i, 0))


def _tc_mlp1(lf, h, s, W64, wh, ws, b1):
  def body(lf_ref, h_ref, s_ref, w_ref, wh_ref, ws_ref, b_ref, y_ref, st_ref):
    i = pl.program_id(0)
    y = jnp.dot(lf_ref[...], w_ref[...].T, preferred_element_type=jnp.float32)
    y = y + h_ref[...] * wh_ref[...] + s_ref[...] * ws_ref[...] + b_ref[...]
    y_ref[...] = y
    _acc_stats(i, y, st_ref)

  return pl.pallas_call(
      body,
      grid=(_NB,),
      in_specs=[
          _row_spec(64), _row_spec(1), _row_spec(1),
          _whole((32, 64)), _whole((1, 32)), _whole((1, 32)), _whole((1, 32)),
      ],
      out_specs=[_row_spec(32), _whole((2, 32))],
      out_shape=[
          jax.ShapeDtypeStruct((_N, 32), jnp.float32),
          jax.ShapeDtypeStruct((2, 32), jnp.float32),
      ],
  )(lf, h, s, W64, wh, ws, b1)


def _tc_mid(y_in, st_in, g, bbn, W, b, din, dout):
  def body(y_ref, st_ref_in, g_ref, bbn_ref, w_ref, b_ref, y_ref_out, st_ref):
    i = pl.program_id(0)
    m, v = _mv(st_ref_in[...])
    x = _bn_act(y_ref[...], m, v, g_ref[...], bbn_ref[...])
    y = jnp.dot(x, w_ref[...].T, preferred_element_type=jnp.float32) + b_ref[...]
    y_ref_out[...] = y
    _acc_stats(i, y, st_ref)

  return pl.pallas_call(
      body,
      grid=(_NB,),
      in_specs=[
          _row_spec(din),
          _whole((2, din)), _whole((1, din)), _whole((1, din)),
          _whole((dout, din)), _whole((1, dout)),
      ],
      out_specs=[_row_spec(dout), _whole((2, dout))],
      out_shape=[
          jax.ShapeDtypeStruct((_N, dout), jnp.float32),
          jax.ShapeDtypeStruct((2, dout), jnp.float32),
      ],
  )(y_in, st_in, g, bbn, W, b)


def _tc_bnact_pad(y_in, st_in, g, bbn):
  def body(y_ref, st_ref, g_ref, b_ref, x_ref):
    m, v = _mv(st_ref[...])
    x_ref[...] = _bn_act(y_ref[...], m, v, g_ref[...], b_ref[...])

  return pl.pallas_call(
      body,
      grid=(_NB,),
      in_specs=[
          _row_spec(8),
          _whole((2, 8)), _whole((1, 8)), _whole((1, 8)),
      ],
      out_specs=_row_spec(8),
      out_shape=jax.ShapeDtypeStruct((_NT, 8), jnp.float32),
  )(y_in, st_in, g, bbn)


def _tc_combine_s(agg2, recip8, x, Wlbig, blbig, Wrbig):
  def body(a_ref, r_ref, x_ref, wl_ref, bl_ref, wr_ref, z_ref, st_ref):
    mean = (a_ref[0] + a_ref[1]) * r_ref[...]
    z = (jnp.dot(mean, wl_ref[...], preferred_element_type=jnp.float32)
         + jnp.dot(x_ref[...], wr_ref[...], preferred_element_type=jnp.float32)
         + bl_ref[...])
    z_ref[...] = z
    rowid = lax.broadcasted_iota(jnp.int32, (_SROWS, 128), 0)
    zm = jnp.where(rowid < _VROWS, z, 0.0)
    st_ref[...] = jnp.stack([jnp.sum(zm, 0), jnp.sum(zm * zm, 0)])

  return pl.pallas_call(
      body,
      out_shape=[
          jax.ShapeDtypeStruct((_SROWS, 128), jnp.float32),
          jax.ShapeDtypeStruct((2, 128), jnp.float32),
      ],
  )(agg2, recip8, x, Wlbig, blbig, Wrbig)


def _tc_bnact_s(z, st, S, g128, b128):
  def body(z_ref, st_ref, s_ref, g_ref, b_ref, x_ref):
    ss = jnp.dot(st_ref[...], s_ref[...], preferred_element_type=jnp.float32)
    m = ss[0] * (1.0 / _N)
    v = ss[1] * (1.0 / _N) - m * m
    x_ref[...] = _bn_act(z_ref[...], m[None, :], v[None, :], g_ref[...],
                         b_ref[...])

  return pl.pallas_call(
      body,
      out_shape=jax.ShapeDtypeStruct((_SROWS, 128), jnp.float32),
  )(z, st, S, g128, b128)


def _tc_bnact_cls_s(z, st, S, g128, b128, Wcbig, bc48):
  def body(z_ref, st_ref, s_ref, g_ref, b_ref, w_ref, bc_ref, o_ref):
    ss = jnp.dot(st_ref[...], s_ref[...], preferred_element_type=jnp.float32)
    m = ss[0] * (1.0 / _N)
    v = ss[1] * (1.0 / _N) - m * m
    x = _bn_act(z_ref[...], m[None, :], v[None, :], g_ref[...], b_ref[...])
    res = jnp.dot(x, w_ref[...], preferred_element_type=jnp.float32) + bc_ref[...]
    o_ref[...] = res[:_VROWS]

  return pl.pallas_call(
      body,
      out_shape=jax.ShapeDtypeStruct((_VROWS, 48), jnp.float32),
  )(z, st, S, g128, b128, Wcbig, bc48)


def kernel(latent_features, edge_data, history, is_subgoal, params):
  p = params
  lf = latent_features.astype(jnp.float32)
  h = history.astype(jnp.float32).reshape(_N, 1)
  s = is_subgoal.astype(jnp.float32).reshape(_N, 1)

  epad = jnp.pad(edge_data.astype(jnp.int32), ((0, 0), (0, _EHPAD - _EH)),
                 constant_values=_N).reshape(2, _NW, _OUT, _SUP, _CHUNK)
  zeros1 = jnp.zeros((_ZROWS,), jnp.float32)
  zeros8 = jnp.zeros((_ZROWS, 8), jnp.float32)

  recip8 = _sc_recip8(epad, zeros1).reshape(_SROWS, 128)

  lane = jnp.arange(128)
  S = (lane[:, None] % 8 == lane[None, :] % 8).astype(jnp.float32)
  eye16 = jnp.eye(16, dtype=jnp.float32)

  W1 = p['fc1_W']
  y1, st1 = _tc_mlp1(lf, h, s, W1[:, :64],
                     W1[:, 64].reshape(1, 32), W1[:, 65].reshape(1, 32),
                     p['fc1_b'].reshape(1, 32))
  y2, st2 = _tc_mid(y1, st1, p['fc1bn_g'].reshape(1, 32),
                    p['fc1bn_b'].reshape(1, 32), p['fc2_W'],
                    p['fc2_b'].reshape(1, 16), 32, 16)
  y3, st3 = _tc_mid(y2, st2, p['fc2bn_g'].reshape(1, 16),
                    p['fc2bn_b'].reshape(1, 16), p['fc3_W'],
                    p['fc3_b'].reshape(1, 8), 16, 8)
  xt = _tc_bnact_pad(y3, st3, p['fc3bn_g'].reshape(1, 8),
                     p['fc3bn_b'].reshape(1, 8))
  xts = xt.reshape(_SROWS, 128)

  z, stz = None, None
  for k in (1, 2, 3):
    agg_parts = _sc_agg(xts.reshape(_NT, 8), epad, zeros8)
    agg2 = agg_parts.reshape(_NC, _SROWS, 128)
    Wlbig = jnp.kron(eye16, p['conv%d_Wl' % k].T)
    Wrbig = jnp.kron(eye16, p['conv%d_Wr' % k].T)
    blbig = jnp.tile(p['conv%d_bl' % k], 16).reshape(1, 128)
    z, stz = _tc_combine_s(agg2, recip8, xts, Wlbig, blbig, Wrbig)
    if k < 3:
      g128 = jnp.tile(p['conv%dbn_g' % k], 16).reshape(1, 128)
      b128 = jnp.tile(p['conv%dbn_b' % k], 16).reshape(1, 128)
      xts = _tc_bnact_s(z, stz, S, g128, b128)

  g128 = jnp.tile(p['conv3bn_g'], 16).reshape(1, 128)
  b128 = jnp.tile(p['conv3bn_b'], 16).reshape(1, 128)
  Wcbig = jnp.kron(eye16, p['cls_W'].T)
  bc48 = jnp.tile(p['cls_b'], 16).reshape(1, 48)
  out = _tc_bnact_cls_s(z, stz, S, g128, b128, Wcbig, bc48)
  return out.reshape(_N, 3)


# speedup vs baseline: 24.2104x; 1.0858x over previous
import jax
import jax.numpy as jnp
from jax import lax
from jax.experimental import pallas as pl
from jax.experimental.pallas import tpu as pltpu
from jax.experimental.pallas import tpu_sc as plsc

_N = 100000
_EH = 800000
_NC, _NS = 2, 16
_NW = _NC * _NS
_CHUNK = 128
_GRP = 4
_SUP = 28
_OUT = 7
_PERW = _CHUNK * _SUP * _OUT
_EHPAD = _NW * _PERW
_EROWS = _EHPAD // _CHUNK
_BROWS = _SUP * _OUT
_NT = 100096
_ZROWS = _NT // _NS
_SROWS = _NT // 16
_VROWS = _N // 16
_BR = 2000
_NB = _N // _BR
_EPS = 1e-5
_SLOPE = 0.1

_mesh = plsc.VectorSubcoreMesh(core_axis_name="c", subcore_axis_name="s")
_sc_params = pltpu.CompilerParams(use_tc_tiling_on_sc=False,
                                  needs_layout_passes=False)


def _sc_recip8(epad, zeros1):
  def body(e_hbm, zeros_hbm, out_hbm, ia_v, ib_v, ones_v, zcnt_v, rexp_v,
           cnt_sp, sem):
    cid = lax.axis_index("c")
    sid = lax.axis_index("s")

    @pl.when(cid == 0)
    def _():
      for i in range(_CHUNK // 16):
        ones_v[pl.ds(i * 16, 16)] = jnp.ones((16,), jnp.float32)
      pltpu.sync_copy(zeros_hbm, zcnt_v)
      pltpu.sync_copy(zcnt_v, cnt_sp.at[pl.ds(sid * _ZROWS, _ZROWS)])
      plsc.subcore_barrier()

      def outer(o, _):
        blk = sid * 2 + o // _OUT
        oo = o % _OUT
        rowbase = blk * _BROWS + oo * _SUP
        pltpu.sync_copy(e_hbm.at[0, pl.ds(rowbase, _SUP)], ia_v)
        pltpu.sync_copy(e_hbm.at[1, pl.ds(rowbase, _SUP)], ib_v)

        def inner(p, _):
          j0 = p * _GRP
          sds = []
          for b in range(_GRP):
            for k in range(_CHUNK // 16):
              iva = ia_v[j0 + b, pl.ds(k * 16, 16)]
              sds.append(pltpu.async_copy(
                  ones_v.at[pl.ds(0, 16)], cnt_sp.at[iva], sem, add=True))
              ivb = ib_v[j0 + b, pl.ds(k * 16, 16)]
              sds.append(pltpu.async_copy(
                  ones_v.at[pl.ds(0, 16)], cnt_sp.at[ivb], sem, add=True))
          for sd in sds:
            sd.wait()
          return 0

        lax.fori_loop(0, _SUP // _GRP, inner, 0)
        return 0

      lax.fori_loop(0, 2 * _OUT, outer, 0)
      plsc.subcore_barrier()
      pltpu.sync_copy(cnt_sp.at[pl.ds(sid * _ZROWS, _ZROWS)], zcnt_v)

      def expand(i, _):
        c = zcnt_v[pl.ds(i * 16, 16)]
        r = 1.0 / jnp.maximum(c, 1.0)
        rows = i * 16 + lax.iota(jnp.int32, 16)
        for f in range(8):
          plsc.store_scatter(rexp_v, [rows, jnp.full((16,), f, jnp.int32)], r)
        return 0

      lax.fori_loop(0, _ZROWS // 16, expand, 0)
      pltpu.sync_copy(rexp_v, out_hbm.at[sid])

  f = pl.kernel(
      body,
      out_type=jax.ShapeDtypeStruct((_NS, _ZROWS, 8), jnp.float32),
      mesh=_mesh,
      compiler_params=_sc_params,
      scratch_types=[
          pltpu.VMEM((_SUP, _CHUNK), jnp.int32),
          pltpu.VMEM((_SUP, _CHUNK), jnp.int32),
          pltpu.VMEM((_CHUNK,), jnp.float32),
          pltpu.VMEM((_ZROWS,), jnp.float32),
          pltpu.VMEM((_ZROWS, 8), jnp.float32),
          pltpu.VMEM_SHARED((_NT,), jnp.float32),
          pltpu.SemaphoreType.DMA,
      ],
  )
  return f(epad, zeros1)


def _sc_agg(xt, epad, zeros8):
  def body(x_hbm, e_hbm, zeros_hbm, out_hbm,
           ia_v, ib_v, rows_v, zrow_v, agg_sp, gsem, ssem):
    cid = lax.axis_index("c")
    sid = lax.axis_index("s")
    wid = sid * _NC + cid
    pltpu.sync_copy(zeros_hbm, zrow_v)
    pltpu.sync_copy(zrow_v, agg_sp.at[pl.ds(sid * _ZROWS, _ZROWS)])
    plsc.subcore_barrier()

    def outer(o, _):
      rowbase = wid * _BROWS + o * _SUP
      pltpu.sync_copy(e_hbm.at[0, pl.ds(rowbase, _SUP)], ia_v)
      pltpu.sync_copy(e_hbm.at[1, pl.ds(rowbase, _SUP)], ib_v)

      def inner(p, _):
        j0 = p * _GRP
        gds = []
        for b in range(_GRP):
          gds.append(pltpu.async_copy(
              x_hbm.at[ia_v.at[j0 + b]], rows_v.at[b], gsem))
          gds.append(pltpu.async_copy(
              x_hbm.at[ib_v.at[j0 + b]], rows_v.at[_GRP + b], gsem))
        sds = []
        for b in range(_GRP):
          gds[2 * b].wait()
          for k in range(_CHUNK // 16):
            ivb = ib_v[j0 + b, pl.ds(k * 16, 16)]
            sds.append(pltpu.async_copy(
                rows_v.at[b, pl.ds(k * 16, 16)], agg_sp.at[ivb], ssem,
                add=True))
          gds[2 * b + 1].wait()
          for k in range(_CHUNK // 16):
            iva = ia_v[j0 + b, pl.ds(k * 16, 16)]
            sds.append(pltpu.async_copy(
                rows_v.at[_GRP + b, pl.ds(k * 16, 16)], agg_sp.at[iva], ssem,
                add=True))
        for sd in sds:
          sd.wait()
        return 0

      lax.fori_loop(0, _SUP // _GRP, inner, 0)
      return 0

    lax.fori_loop(0, _OUT, outer, 0)
    plsc.subcore_barrier()
    pltpu.sync_copy(agg_sp.at[pl.ds(sid * _ZROWS, _ZROWS)], zrow_v)
    pltpu.sync_copy(zrow_v, out_hbm.at[cid, sid])

  f = pl.kernel(
      body,
      out_type=jax.ShapeDtypeStruct((_NC, _NS, _ZROWS, 8), jnp.float32),
      mesh=_mesh,
      compiler_params=_sc_params,
      scratch_types=[
          pltpu.VMEM((_SUP, _CHUNK), jnp.int32),
          pltpu.VMEM((_SUP, _CHUNK), jnp.int32),
          pltpu.VMEM((2 * _GRP, _CHUNK, 8), jnp.float32),
          pltpu.VMEM((_ZROWS, 8), jnp.float32),
          pltpu.VMEM_SHARED((_NT, 8), jnp.float32),
          pltpu.SemaphoreType.DMA,
          pltpu.SemaphoreType.DMA,
      ],
  )
  return f(xt, epad, zeros8)


def _acc_stats(i, y, st_ref):
  part = jnp.stack([jnp.sum(y, 0), jnp.sum(y * y, 0)])

  @pl.when(i == 0)
  def _():
    st_ref[...] = part

  @pl.when(i > 0)
  def _():
    st_ref[...] += part


def _bn_act(y, m, v, g, b):
  xh = (y - m) * lax.rsqrt(v + _EPS) * g + b
  return jnp.where(xh >= 0, xh, _SLOPE * xh)


def _b16(x):
  return x.astype(jnp.bfloat16)


def _mv(st):
  m = st[0] * (1.0 / _N)
  v = st[1] * (1.0 / _N) - m * m
  return m[None, :], v[None, :]


def _whole(shape):
  return pl.BlockSpec(shape, lambda i: tuple(0 for _ in shape))


def _row_spec(d):
  return pl.BlockSpec((_BR, d), lambda i: (i, 0))


def _tc_mlp1(lf, h, s, W64, wh, ws, b1):
  def body(lf_ref, h_ref, s_ref, w_ref, wh_ref, ws_ref, b_ref, y_ref, st_ref):
    i = pl.program_id(0)
    y = jnp.dot(_b16(lf_ref[...]), _b16(w_ref[...].T),
                preferred_element_type=jnp.float32)
    hc = _b16(h_ref[...]).astype(jnp.float32) * _b16(wh_ref[...]).astype(jnp.float32)
    sc = _b16(s_ref[...]).astype(jnp.float32) * _b16(ws_ref[...]).astype(jnp.float32)
    y = y + hc + sc + b_ref[...]
    y_ref[...] = y
    _acc_stats(i, y, st_ref)

  return pl.pallas_call(
      body,
      grid=(_NB,),
      in_specs=[
          _row_spec(64), _row_spec(1), _row_spec(1),
          _whole((32, 64)), _whole((1, 32)), _whole((1, 32)), _whole((1, 32)),
      ],
      out_specs=[_row_spec(32), _whole((2, 32))],
      out_shape=[
          jax.ShapeDtypeStruct((_N, 32), jnp.float32),
          jax.ShapeDtypeStruct((2, 32), jnp.float32),
      ],
  )(lf, h, s, W64, wh, ws, b1)


def _tc_mid(y_in, st_in, g, bbn, W, b, din, dout):
  def body(y_ref, st_ref_in, g_ref, bbn_ref, w_ref, b_ref, y_ref_out, st_ref):
    i = pl.program_id(0)
    m, v = _mv(st_ref_in[...])
    x = _bn_act(y_ref[...], m, v, g_ref[...], bbn_ref[...])
    y = jnp.dot(_b16(x), _b16(w_ref[...].T),
                preferred_element_type=jnp.float32) + b_ref[...]
    y_ref_out[...] = y
    _acc_stats(i, y, st_ref)

  return pl.pallas_call(
      body,
      grid=(_NB,),
      in_specs=[
          _row_spec(din),
          _whole((2, din)), _whole((1, din)), _whole((1, din)),
          _whole((dout, din)), _whole((1, dout)),
      ],
      out_specs=[_row_spec(dout), _whole((2, dout))],
      out_shape=[
          jax.ShapeDtypeStruct((_N, dout), jnp.float32),
          jax.ShapeDtypeStruct((2, dout), jnp.float32),
      ],
  )(y_in, st_in, g, bbn, W, b)


def _tc_bnact_s6(y_s, st, T, g128, b128):
  def body(y_ref, st_ref, t_ref, g_ref, b_ref, x_ref):
    ss = jnp.dot(st_ref[...], t_ref[...], preferred_element_type=jnp.float32)
    m = ss[0] * (1.0 / _N)
    v = ss[1] * (1.0 / _N) - m * m
    x = _bn_act(y_ref[...], m[None, :], v[None, :], g_ref[...], b_ref[...])
    x_ref[...] = jnp.concatenate(
        [x, jnp.zeros((_SROWS - _VROWS, 128), jnp.float32)], axis=0)

  return pl.pallas_call(
      body,
      out_shape=jax.ShapeDtypeStruct((_SROWS, 128), jnp.float32),
  )(y_s, st, T, g128, b128)

def _tc_combine_s(agg2, recip8, x, Wlbig, blbig, Wrbig):
  def body(a_ref, r_ref, x_ref, wl_ref, bl_ref, wr_ref, z_ref, st_ref):
    mean = (a_ref[0] + a_ref[1]) * r_ref[...]
    z = (jnp.dot(_b16(mean), _b16(wl_ref[...]),
                 preferred_element_type=jnp.float32)
         + jnp.dot(_b16(x_ref[...]), _b16(wr_ref[...]),
                   preferred_element_type=jnp.float32)
         + bl_ref[...])
    z_ref[...] = z
    rowid = lax.broadcasted_iota(jnp.int32, (_SROWS, 128), 0)
    zm = jnp.where(rowid < _VROWS, z, 0.0)
    st_ref[...] = jnp.stack([jnp.sum(zm, 0), jnp.sum(zm * zm, 0)])

  return pl.pallas_call(
      body,
      out_shape=[
          jax.ShapeDtypeStruct((_SROWS, 128), jnp.float32),
          jax.ShapeDtypeStruct((2, 128), jnp.float32),
      ],
  )(agg2, recip8, x, Wlbig, blbig, Wrbig)


def _tc_bnact_s(z, st, S, g128, b128):
  def body(z_ref, st_ref, s_ref, g_ref, b_ref, x_ref):
    ss = jnp.dot(st_ref[...], s_ref[...], preferred_element_type=jnp.float32)
    m = ss[0] * (1.0 / _N)
    v = ss[1] * (1.0 / _N) - m * m
    x_ref[...] = _bn_act(z_ref[...], m[None, :], v[None, :], g_ref[...],
                         b_ref[...])

  return pl.pallas_call(
      body,
      out_shape=jax.ShapeDtypeStruct((_SROWS, 128), jnp.float32),
  )(z, st, S, g128, b128)


def _tc_bnact_cls_s(z, st, S, g128, b128, Wcbig, bc48):
  def body(z_ref, st_ref, s_ref, g_ref, b_ref, w_ref, bc_ref, o_ref):
    ss = jnp.dot(st_ref[...], s_ref[...], preferred_element_type=jnp.float32)
    m = ss[0] * (1.0 / _N)
    v = ss[1] * (1.0 / _N) - m * m
    x = _bn_act(z_ref[...], m[None, :], v[None, :], g_ref[...], b_ref[...])
    res = jnp.dot(_b16(x), _b16(w_ref[...]),
                  preferred_element_type=jnp.float32) + bc_ref[...]
    o_ref[...] = res[:_VROWS]

  return pl.pallas_call(
      body,
      out_shape=jax.ShapeDtypeStruct((_VROWS, 48), jnp.float32),
  )(z, st, S, g128, b128, Wcbig, bc48)


def kernel(latent_features, edge_data, history, is_subgoal, params):
  p = params
  lf = latent_features.astype(jnp.float32)
  h = history.astype(jnp.float32).reshape(_N, 1)
  s = is_subgoal.astype(jnp.float32).reshape(_N, 1)

  epad = jnp.pad(edge_data.astype(jnp.int32), ((0, 0), (0, _EHPAD - _EH)),
                 constant_values=_N).reshape(2, _EROWS, _CHUNK)
  zeros1 = jnp.zeros((_ZROWS,), jnp.float32)
  zeros8 = jnp.zeros((_ZROWS, 8), jnp.float32)

  recip8 = _sc_recip8(epad, zeros1).reshape(_SROWS, 128)

  lane = jnp.arange(128)
  S = (lane[:, None] % 8 == lane[None, :] % 8).astype(jnp.float32)
  T = (jnp.arange(8)[:, None] == lane[None, :] % 8).astype(jnp.float32)
  eye16 = jnp.eye(16, dtype=jnp.float32)

  W1 = p['fc1_W']
  y1, st1 = _tc_mlp1(lf, h, s, W1[:, :64],
                     W1[:, 64].reshape(1, 32), W1[:, 65].reshape(1, 32),
                     p['fc1_b'].reshape(1, 32))
  y2, st2 = _tc_mid(y1, st1, p['fc1bn_g'].reshape(1, 32),
                    p['fc1bn_b'].reshape(1, 32), p['fc2_W'],
                    p['fc2_b'].reshape(1, 16), 32, 16)
  y3, st3 = _tc_mid(y2, st2, p['fc2bn_g'].reshape(1, 16),
                    p['fc2bn_b'].reshape(1, 16), p['fc3_W'],
                    p['fc3_b'].reshape(1, 8), 16, 8)
  st3 = st3 + 0.0 * recip8[0, 0]
  xts = _tc_bnact_s6(y3.reshape(_VROWS, 128), st3, T,
                     jnp.tile(p['fc3bn_g'], 16).reshape(1, 128),
                     jnp.tile(p['fc3bn_b'], 16).reshape(1, 128))

  z, stz = None, None
  for k in (1, 2, 3):
    agg_parts = _sc_agg(xts.reshape(_NT, 8), epad, zeros8)
    agg2 = agg_parts.reshape(_NC, _SROWS, 128)
    Wlbig = jnp.kron(eye16, p['conv%d_Wl' % k].T)
    Wrbig = jnp.kron(eye16, p['conv%d_Wr' % k].T)
    blbig = jnp.tile(p['conv%d_bl' % k], 16).reshape(1, 128)
    z, stz = _tc_combine_s(agg2, recip8, xts, Wlbig, blbig, Wrbig)
    if k < 3:
      g128 = jnp.tile(p['conv%dbn_g' % k], 16).reshape(1, 128)
      b128 = jnp.tile(p['conv%dbn_b' % k], 16).reshape(1, 128)
      xts = _tc_bnact_s(z, stz, S, g128, b128)

  g128 = jnp.tile(p['conv3bn_g'], 16).reshape(1, 128)
  b128 = jnp.tile(p['conv3bn_b'], 16).reshape(1, 128)
  Wcbig = jnp.kron(eye16, p['cls_W'].T)
  bc48 = jnp.tile(p['cls_b'], 16).reshape(1, 48)
  out = _tc_bnact_cls_s(z, stz, S, g128, b128, Wcbig, bc48)
  return out.reshape(_N, 3)


# speedup vs baseline: 24.7538x; 1.0224x over previous
import jax
import jax.numpy as jnp
from jax import lax
from jax.experimental import pallas as pl
from jax.experimental.pallas import tpu as pltpu
from jax.experimental.pallas import tpu_sc as plsc

_N = 100000
_EH = 800000
_NC, _NS = 2, 16
_NW = _NC * _NS
_CHUNK = 128
_GRP = 4
_SUP = 28
_OUT = 7
_PERW = _CHUNK * _SUP * _OUT
_EHPAD = _NW * _PERW
_EROWS = _EHPAD // _CHUNK
_BROWS = _SUP * _OUT
_NT = 100096
_ZROWS = _NT // _NS
_SROWS = _NT // 16
_VROWS = _N // 16
_BR = 2000
_NB = _N // _BR
_EPS = 1e-5
_SLOPE = 0.1

_mesh = plsc.VectorSubcoreMesh(core_axis_name="c", subcore_axis_name="s")
_sc_params = pltpu.CompilerParams(use_tc_tiling_on_sc=False,
                                  needs_layout_passes=False)


def _sc_recip8(epad, zeros1):
  def body(e_hbm, zeros_hbm, out_hbm, ia_v, ib_v, ones_v, zcnt_v, rexp_v,
           cnt_sp, sem):
    cid = lax.axis_index("c")
    sid = lax.axis_index("s")

    @pl.when(cid == 0)
    def _():
      for i in range(_CHUNK // 16):
        ones_v[pl.ds(i * 16, 16)] = jnp.ones((16,), jnp.float32)
      pltpu.sync_copy(zeros_hbm, zcnt_v)
      pltpu.sync_copy(zcnt_v, cnt_sp.at[pl.ds(sid * _ZROWS, _ZROWS)])
      plsc.subcore_barrier()

      def outer(o, _):
        blk = sid * 2 + o // _OUT
        oo = o % _OUT
        rowbase = blk * _BROWS + oo * _SUP
        pltpu.sync_copy(e_hbm.at[0, pl.ds(rowbase, _SUP)], ia_v)
        pltpu.sync_copy(e_hbm.at[1, pl.ds(rowbase, _SUP)], ib_v)

        def inner(p, _):
          j0 = p * _GRP
          sds = []
          for b in range(_GRP):
            sds.append(pltpu.async_copy(
                ones_v, cnt_sp.at[ia_v.at[j0 + b]], sem, add=True))
            sds.append(pltpu.async_copy(
                ones_v, cnt_sp.at[ib_v.at[j0 + b]], sem, add=True))
          for sd in sds:
            sd.wait()
          return 0

        lax.fori_loop(0, _SUP // _GRP, inner, 0)
        return 0

      lax.fori_loop(0, 2 * _OUT, outer, 0)
      plsc.subcore_barrier()
      pltpu.sync_copy(cnt_sp.at[pl.ds(sid * _ZROWS, _ZROWS)], zcnt_v)

      def expand(i, _):
        c = zcnt_v[pl.ds(i * 16, 16)]
        r = 1.0 / jnp.maximum(c, 1.0)
        rows = i * 16 + lax.iota(jnp.int32, 16)
        for f in range(8):
          plsc.store_scatter(rexp_v, [rows, jnp.full((16,), f, jnp.int32)], r)
        return 0

      lax.fori_loop(0, _ZROWS // 16, expand, 0)
      pltpu.sync_copy(rexp_v, out_hbm.at[sid])

  f = pl.kernel(
      body,
      out_type=jax.ShapeDtypeStruct((_NS, _ZROWS, 8), jnp.float32),
      mesh=_mesh,
      compiler_params=_sc_params,
      scratch_types=[
          pltpu.VMEM((_SUP, _CHUNK), jnp.int32),
          pltpu.VMEM((_SUP, _CHUNK), jnp.int32),
          pltpu.VMEM((_CHUNK,), jnp.float32),
          pltpu.VMEM((_ZROWS,), jnp.float32),
          pltpu.VMEM((_ZROWS, 8), jnp.float32),
          pltpu.VMEM_SHARED((_NT,), jnp.float32),
          pltpu.SemaphoreType.DMA,
      ],
  )
  return f(epad, zeros1)


def _sc_agg(xt, epad, zeros8):
  def body(x_hbm, e_hbm, zeros_hbm, out_hbm,
           ia_v, ib_v, rows_v, zrow_v, agg_sp, gsem, ssem):
    cid = lax.axis_index("c")
    sid = lax.axis_index("s")
    wid = sid * _NC + cid
    pltpu.sync_copy(zeros_hbm, zrow_v)
    pltpu.sync_copy(zrow_v, agg_sp.at[pl.ds(sid * _ZROWS, _ZROWS)])
    plsc.subcore_barrier()

    def outer(o, _):
      rowbase = wid * _BROWS + o * _SUP
      pltpu.sync_copy(e_hbm.at[0, pl.ds(rowbase, _SUP)], ia_v)
      pltpu.sync_copy(e_hbm.at[1, pl.ds(rowbase, _SUP)], ib_v)

      def inner(p, _):
        j0 = p * _GRP
        gds = []
        for b in range(_GRP):
          gds.append(pltpu.async_copy(
              x_hbm.at[ia_v.at[j0 + b]], rows_v.at[b], gsem))
          gds.append(pltpu.async_copy(
              x_hbm.at[ib_v.at[j0 + b]], rows_v.at[_GRP + b], gsem))
        sds = []
        for b in range(_GRP):
          gds[2 * b].wait()
          sds.append(pltpu.async_copy(
              rows_v.at[b], agg_sp.at[ib_v.at[j0 + b]], ssem, add=True))
          gds[2 * b + 1].wait()
          sds.append(pltpu.async_copy(
              rows_v.at[_GRP + b], agg_sp.at[ia_v.at[j0 + b]], ssem,
              add=True))
        for sd in sds:
          sd.wait()
        return 0

      lax.fori_loop(0, _SUP // _GRP, inner, 0)
      return 0

    lax.fori_loop(0, _OUT, outer, 0)
    plsc.subcore_barrier()
    pltpu.sync_copy(agg_sp.at[pl.ds(sid * _ZROWS, _ZROWS)], zrow_v)
    pltpu.sync_copy(zrow_v, out_hbm.at[cid, sid])

  f = pl.kernel(
      body,
      out_type=jax.ShapeDtypeStruct((_NC, _NS, _ZROWS, 8), jnp.float32),
      mesh=_mesh,
      compiler_params=_sc_params,
      scratch_types=[
          pltpu.VMEM((_SUP, _CHUNK), jnp.int32),
          pltpu.VMEM((_SUP, _CHUNK), jnp.int32),
          pltpu.VMEM((2 * _GRP, _CHUNK, 8), jnp.float32),
          pltpu.VMEM((_ZROWS, 8), jnp.float32),
          pltpu.VMEM_SHARED((_NT, 8), jnp.float32),
          pltpu.SemaphoreType.DMA,
          pltpu.SemaphoreType.DMA,
      ],
  )
  return f(xt, epad, zeros8)


def _acc_stats(i, y, st_ref):
  part = jnp.stack([jnp.sum(y, 0), jnp.sum(y * y, 0)])

  @pl.when(i == 0)
  def _():
    st_ref[...] = part

  @pl.when(i > 0)
  def _():
    st_ref[...] += part


def _bn_act(y, m, v, g, b):
  xh = (y - m) * lax.rsqrt(v + _EPS) * g + b
  return jnp.where(xh >= 0, xh, _SLOPE * xh)


def _b16(x):
  return x.astype(jnp.bfloat16)


def _mv(st):
  m = st[0] * (1.0 / _N)
  v = st[1] * (1.0 / _N) - m * m
  return m[None, :], v[None, :]


def _whole(shape):
  return pl.BlockSpec(shape, lambda i: tuple(0 for _ in shape))


def _row_spec(d):
  return pl.BlockSpec((_BR, d), lambda i: (i, 0))


def _tc_mlp1(lf, h, s, W64, wh, ws, b1):
  def body(lf_ref, h_ref, s_ref, w_ref, wh_ref, ws_ref, b_ref, y_ref, st_ref):
    i = pl.program_id(0)
    y = jnp.dot(_b16(lf_ref[...]), _b16(w_ref[...].T),
                preferred_element_type=jnp.float32)
    hc = _b16(h_ref[...]).astype(jnp.float32) * _b16(wh_ref[...]).astype(jnp.float32)
    sc = _b16(s_ref[...]).astype(jnp.float32) * _b16(ws_ref[...]).astype(jnp.float32)
    y = y + hc + sc + b_ref[...]
    y_ref[...] = y
    _acc_stats(i, y, st_ref)

  return pl.pallas_call(
      body,
      grid=(_NB,),
      in_specs=[
          _row_spec(64), _row_spec(1), _row_spec(1),
          _whole((32, 64)), _whole((1, 32)), _whole((1, 32)), _whole((1, 32)),
      ],
      out_specs=[_row_spec(32), _whole((2, 32))],
      out_shape=[
          jax.ShapeDtypeStruct((_N, 32), jnp.float32),
          jax.ShapeDtypeStruct((2, 32), jnp.float32),
      ],
  )(lf, h, s, W64, wh, ws, b1)


def _tc_mid(y_in, st_in, g, bbn, W, b, din, dout):
  def body(y_ref, st_ref_in, g_ref, bbn_ref, w_ref, b_ref, y_ref_out, st_ref):
    i = pl.program_id(0)
    m, v = _mv(st_ref_in[...])
    x = _bn_act(y_ref[...], m, v, g_ref[...], bbn_ref[...])
    y = jnp.dot(_b16(x), _b16(w_ref[...].T),
                preferred_element_type=jnp.float32) + b_ref[...]
    y_ref_out[...] = y
    _acc_stats(i, y, st_ref)

  return pl.pallas_call(
      body,
      grid=(_NB,),
      in_specs=[
          _row_spec(din),
          _whole((2, din)), _whole((1, din)), _whole((1, din)),
          _whole((dout, din)), _whole((1, dout)),
      ],
      out_specs=[_row_spec(dout), _whole((2, dout))],
      out_shape=[
          jax.ShapeDtypeStruct((_N, dout), jnp.float32),
          jax.ShapeDtypeStruct((2, dout), jnp.float32),
      ],
  )(y_in, st_in, g, bbn, W, b)


def _tc_bnact_s6(y_s, st, T, g128, b128):
  def body(y_ref, st_ref, t_ref, g_ref, b_ref, x_ref):
    ss = jnp.dot(st_ref[...], t_ref[...], preferred_element_type=jnp.float32)
    m = ss[0] * (1.0 / _N)
    v = ss[1] * (1.0 / _N) - m * m
    x = _bn_act(y_ref[...], m[None, :], v[None, :], g_ref[...], b_ref[...])
    x_ref[...] = jnp.concatenate(
        [x, jnp.zeros((_SROWS - _VROWS, 128), jnp.float32)], axis=0)

  return pl.pallas_call(
      body,
      out_shape=jax.ShapeDtypeStruct((_SROWS, 128), jnp.float32),
  )(y_s, st, T, g128, b128)

def _tc_combine_s(agg2, recip8, x, Wlbig, blbig, Wrbig):
  def body(a_ref, r_ref, x_ref, wl_ref, bl_ref, wr_ref, z_ref, st_ref):
    mean = (a_ref[0] + a_ref[1]) * r_ref[...]
    z = (jnp.dot(_b16(mean), _b16(wl_ref[...]),
                 preferred_element_type=jnp.float32)
         + jnp.dot(_b16(x_ref[...]), _b16(wr_ref[...]),
                   preferred_element_type=jnp.float32)
         + bl_ref[...])
    z_ref[...] = z
    rowid = lax.broadcasted_iota(jnp.int32, (_SROWS, 128), 0)
    zm = jnp.where(rowid < _VROWS, z, 0.0)
    st_ref[...] = jnp.stack([jnp.sum(zm, 0), jnp.sum(zm * zm, 0)])

  return pl.pallas_call(
      body,
      out_shape=[
          jax.ShapeDtypeStruct((_SROWS, 128), jnp.float32),
          jax.ShapeDtypeStruct((2, 128), jnp.float32),
      ],
  )(agg2, recip8, x, Wlbig, blbig, Wrbig)


def _tc_bnact_s(z, st, S, g128, b128):
  def body(z_ref, st_ref, s_ref, g_ref, b_ref, x_ref):
    ss = jnp.dot(st_ref[...], s_ref[...], preferred_element_type=jnp.float32)
    m = ss[0] * (1.0 / _N)
    v = ss[1] * (1.0 / _N) - m * m
    x_ref[...] = _bn_act(z_ref[...], m[None, :], v[None, :], g_ref[...],
                         b_ref[...])

  return pl.pallas_call(
      body,
      out_shape=jax.ShapeDtypeStruct((_SROWS, 128), jnp.float32),
  )(z, st, S, g128, b128)


def _tc_bnact_cls_s(z, st, S, g128, b128, Wcbig, bc48):
  def body(z_ref, st_ref, s_ref, g_ref, b_ref, w_ref, bc_ref, o_ref):
    ss = jnp.dot(st_ref[...], s_ref[...], preferred_element_type=jnp.float32)
    m = ss[0] * (1.0 / _N)
    v = ss[1] * (1.0 / _N) - m * m
    x = _bn_act(z_ref[...], m[None, :], v[None, :], g_ref[...], b_ref[...])
    res = jnp.dot(_b16(x), _b16(w_ref[...]),
                  preferred_element_type=jnp.float32) + bc_ref[...]
    o_ref[...] = res[:_VROWS]

  return pl.pallas_call(
      body,
      out_shape=jax.ShapeDtypeStruct((_VROWS, 48), jnp.float32),
  )(z, st, S, g128, b128, Wcbig, bc48)


def kernel(latent_features, edge_data, history, is_subgoal, params):
  p = params
  lf = latent_features.astype(jnp.float32)
  h = history.astype(jnp.float32).reshape(_N, 1)
  s = is_subgoal.astype(jnp.float32).reshape(_N, 1)

  epad = jnp.pad(edge_data.astype(jnp.int32), ((0, 0), (0, _EHPAD - _EH)),
                 constant_values=_N).reshape(2, _EROWS, _CHUNK)
  zeros1 = jnp.zeros((_ZROWS,), jnp.float32)
  zeros8 = jnp.zeros((_ZROWS, 8), jnp.float32)

  recip8 = _sc_recip8(epad, zeros1).reshape(_SROWS, 128)

  lane = jnp.arange(128)
  S = (lane[:, None] % 8 == lane[None, :] % 8).astype(jnp.float32)
  T = (jnp.arange(8)[:, None] == lane[None, :] % 8).astype(jnp.float32)
  eye16 = jnp.eye(16, dtype=jnp.float32)

  W1 = p['fc1_W']
  y1, st1 = _tc_mlp1(lf, h, s, W1[:, :64],
                     W1[:, 64].reshape(1, 32), W1[:, 65].reshape(1, 32),
                     p['fc1_b'].reshape(1, 32))
  y2, st2 = _tc_mid(y1, st1, p['fc1bn_g'].reshape(1, 32),
                    p['fc1bn_b'].reshape(1, 32), p['fc2_W'],
                    p['fc2_b'].reshape(1, 16), 32, 16)
  y3, st3 = _tc_mid(y2, st2, p['fc2bn_g'].reshape(1, 16),
                    p['fc2bn_b'].reshape(1, 16), p['fc3_W'],
                    p['fc3_b'].reshape(1, 8), 16, 8)
  st3 = st3 + 0.0 * recip8[0, 0]
  xts = _tc_bnact_s6(y3.reshape(_VROWS, 128), st3, T,
                     jnp.tile(p['fc3bn_g'], 16).reshape(1, 128),
                     jnp.tile(p['fc3bn_b'], 16).reshape(1, 128))

  z, stz = None, None
  for k in (1, 2, 3):
    agg_parts = _sc_agg(xts.reshape(_NT, 8), epad, zeros8)
    agg2 = agg_parts.reshape(_NC, _SROWS, 128)
    Wlbig = jnp.kron(eye16, p['conv%d_Wl' % k].T)
    Wrbig = jnp.kron(eye16, p['conv%d_Wr' % k].T)
    blbig = jnp.tile(p['conv%d_bl' % k], 16).reshape(1, 128)
    z, stz = _tc_combine_s(agg2, recip8, xts, Wlbig, blbig, Wrbig)
    if k < 3:
      g128 = jnp.tile(p['conv%dbn_g' % k], 16).reshape(1, 128)
      b128 = jnp.tile(p['conv%dbn_b' % k], 16).reshape(1, 128)
      xts = _tc_bnact_s(z, stz, S, g128, b128)

  g128 = jnp.tile(p['conv3bn_g'], 16).reshape(1, 128)
  b128 = jnp.tile(p['conv3bn_b'], 16).reshape(1, 128)
  Wcbig = jnp.kron(eye16, p['cls_W'].T)
  bc48 = jnp.tile(p['cls_b'], 16).reshape(1, 48)
  out = _tc_bnact_cls_s(z, stz, S, g128, b128, Wcbig, bc48)
  return out.reshape(_N, 3)
